# Initial kernel scaffold; baseline (speedup 1.0000x reference)
import functools
import math

import jax
import jax.numpy as jnp
from jax import lax
from jax.experimental import pallas as pl
from jax.experimental.pallas import tpu as pltpu
from jax.experimental.pallas import tpu_sc as plsc

VOCAB = 100000
EMBED = 128
B = 1024
L = 200
LANES = 16
CHUNKS_PER_ROW = EMBED // LANES


def _positional_embedding(seq_len, d_model):
    position = jnp.arange(0, seq_len, dtype=jnp.float32)[:, None]
    div_term = jnp.exp(
        jnp.arange(0, d_model, 2, dtype=jnp.float32) * -(math.log(10000.0) / d_model)
    )
    pe = jnp.zeros((seq_len, d_model), dtype=jnp.float32)
    pe = pe.at[:, 0::2].set(jnp.sin(position * div_term))
    pe = pe.at[:, 1::2].set(jnp.cos(position * div_term))
    return pe


def _make_sc_kernel(n_workers, seq_per_w):
    mesh = plsc.VectorSubcoreMesh(core_axis_name="c", subcore_axis_name="s")
    num_cores = mesh.num_cores

    @functools.partial(
        pl.kernel,
        mesh=mesh,
        out_type=jax.ShapeDtypeStruct((B * L, EMBED), jnp.float32),
        scratch_types=[
            pltpu.VMEM((seq_per_w, L), jnp.int32),
            pltpu.VMEM((L, EMBED), jnp.float32),
            pltpu.VMEM((L, EMBED), jnp.float32),
            pltpu.SemaphoreType.DMA,
        ],
    )
    def k(seq_hbm, table_hbm, pe_hbm, out_hbm, idx_v, pe_v, rows_v, sem):
        wid = lax.axis_index("s") * num_cores + lax.axis_index("c")
        seq_base = wid * seq_per_w
        pltpu.sync_copy(pe_hbm, pe_v)
        pltpu.sync_copy(seq_hbm.at[pl.ds(seq_base, seq_per_w)], idx_v)

        def per_seq(si, carry):
            pltpu.async_copy(table_hbm.at[idx_v.at[si]], rows_v, sem).wait()

            def add_row(r, c2):
                for j in range(CHUNKS_PER_ROW):
                    sl = pl.ds(j * LANES, LANES)
                    plsc.addupdate(rows_v.at[r, sl], pe_v[r, sl])
                return c2

            lax.fori_loop(0, L, add_row, 0, unroll=2)

            pltpu.sync_copy(
                rows_v, out_hbm.at[pl.ds((seq_base + si) * L, L)]
            )
            return carry

        lax.fori_loop(0, seq_per_w, per_seq, 0)

    return k


def kernel(sequence, token_table):
    seq = sequence.astype(jnp.int32)
    pe = _positional_embedding(L, EMBED)
    info = plsc.get_sparse_core_info()
    n_workers = info.num_cores * info.num_subcores
    out = _make_sc_kernel(n_workers, B // n_workers)(seq, token_table, pe)
    return out.reshape(B, L, EMBED)


# speedup vs baseline: 1.7679x; 1.7679x over previous
import functools
import math

import jax
import jax.numpy as jnp
from jax import lax
from jax.experimental import pallas as pl
from jax.experimental.pallas import tpu as pltpu
from jax.experimental.pallas import tpu_sc as plsc

VOCAB = 100000
EMBED = 128
B = 1024
L = 200
LANES = 16
CHUNKS_PER_ROW = EMBED // LANES
GCHUNK = 40


def _positional_embedding(seq_len, d_model):
    position = jnp.arange(0, seq_len, dtype=jnp.float32)[:, None]
    div_term = jnp.exp(
        jnp.arange(0, d_model, 2, dtype=jnp.float32) * -(math.log(10000.0) / d_model)
    )
    pe = jnp.zeros((seq_len, d_model), dtype=jnp.float32)
    pe = pe.at[:, 0::2].set(jnp.sin(position * div_term))
    pe = pe.at[:, 1::2].set(jnp.cos(position * div_term))
    return pe


def _make_sc_kernel(n_workers):
    n_chunks = (B * L) // GCHUNK
    chunks_per_w = n_chunks // n_workers
    per_seq = L // GCHUNK
    mesh = plsc.VectorSubcoreMesh(core_axis_name="c", subcore_axis_name="s")
    num_cores = mesh.num_cores

    @functools.partial(
        pl.kernel,
        mesh=mesh,
        out_type=jax.ShapeDtypeStruct((B * L, EMBED), jnp.float32),
        scratch_types=[
            pltpu.VMEM((chunks_per_w, GCHUNK), jnp.int32),
            pltpu.VMEM((L, EMBED), jnp.float32),
            pltpu.VMEM((GCHUNK, EMBED), jnp.float32),
            pltpu.SemaphoreType.DMA,
        ],
    )
    def k(seq_hbm, table_hbm, pe_hbm, out_hbm, idx_v, pe_v, rows_v, sem):
        wid = lax.axis_index("s") * num_cores + lax.axis_index("c")
        chunk_base = wid * chunks_per_w
        pltpu.sync_copy(pe_hbm, pe_v)
        pltpu.sync_copy(seq_hbm.at[pl.ds(chunk_base, chunks_per_w)], idx_v)

        def do_chunk(ci, carry):
            pltpu.async_copy(table_hbm.at[idx_v.at[ci]], rows_v, sem).wait()

            pe_off = lax.rem(ci, per_seq) * GCHUNK

            def add_row(r, c2):
                for j in range(CHUNKS_PER_ROW):
                    sl = pl.ds(j * LANES, LANES)
                    plsc.addupdate(rows_v.at[r, sl], pe_v[pe_off + r, sl])
                return c2

            lax.fori_loop(0, GCHUNK, add_row, 0, unroll=2)

            pltpu.sync_copy(
                rows_v, out_hbm.at[pl.ds((chunk_base + ci) * GCHUNK, GCHUNK)]
            )
            return carry

        lax.fori_loop(0, chunks_per_w, do_chunk, 0)

    return k


def kernel(sequence, token_table):
    n_chunks = (B * L) // GCHUNK
    seq = sequence.astype(jnp.int32).reshape(n_chunks, GCHUNK)
    pe = _positional_embedding(L, EMBED)
    info = plsc.get_sparse_core_info()
    n_workers = info.num_cores * info.num_subcores
    out = _make_sc_kernel(n_workers)(seq, token_table, pe)
    return out.reshape(B, L, EMBED)


# speedup vs baseline: 3.2734x; 1.8516x over previous
import functools
import math

import jax
import jax.numpy as jnp
from jax import lax
from jax.experimental import pallas as pl
from jax.experimental.pallas import tpu as pltpu
from jax.experimental.pallas import tpu_sc as plsc

VOCAB = 100000
EMBED = 128
B = 1024
L = 200
LANES = 16
CHUNKS_PER_ROW = EMBED // LANES
GCHUNK = 40
PE_PERIOD = L // GCHUNK
NBUF = 4
LOOKAHEAD = 2


def _positional_embedding(seq_len, d_model):
    position = jnp.arange(0, seq_len, dtype=jnp.float32)[:, None]
    div_term = jnp.exp(
        jnp.arange(0, d_model, 2, dtype=jnp.float32) * -(math.log(10000.0) / d_model)
    )
    pe = jnp.zeros((seq_len, d_model), dtype=jnp.float32)
    pe = pe.at[:, 0::2].set(jnp.sin(position * div_term))
    pe = pe.at[:, 1::2].set(jnp.cos(position * div_term))
    return pe


def _make_sc_kernel(n_workers):
    n_chunks = (B * L) // GCHUNK
    chunks_per_w = n_chunks // n_workers
    mesh = plsc.VectorSubcoreMesh(core_axis_name="c", subcore_axis_name="s")
    num_cores = mesh.num_cores
    assert (chunks_per_w - 2 * LOOKAHEAD) % NBUF == 0

    @functools.partial(
        pl.kernel,
        mesh=mesh,
        out_type=jax.ShapeDtypeStruct((B * L, EMBED), jnp.float32),
        scratch_types=(
            [pltpu.VMEM((chunks_per_w, GCHUNK), jnp.int32)]
            + [pltpu.VMEM((L, EMBED), jnp.float32)]
            + [pltpu.VMEM((GCHUNK, EMBED), jnp.float32)] * NBUF
            + [pltpu.SemaphoreType.DMA] * (2 * NBUF)
        ),
    )
    def k(seq_hbm, table_hbm, pe_hbm, out_hbm, idx_v, pe_v, *bufs_sems):
        rows = bufs_sems[:NBUF]
        gsem = bufs_sems[NBUF : 2 * NBUF]
        osem = bufs_sems[2 * NBUF :]
        wid = lax.axis_index("s") * num_cores + lax.axis_index("c")
        chunk_base = wid * chunks_per_w
        pltpu.sync_copy(pe_hbm, pe_v)
        pltpu.sync_copy(seq_hbm.at[pl.ds(chunk_base, chunks_per_w)], idx_v)

        def gather_start(c, b):
            pltpu.make_async_copy(
                table_hbm.at[idx_v.at[c]], rows[b], gsem[b]
            ).start()

        def gather_wait(c, b):
            pltpu.make_async_copy(
                table_hbm.at[idx_v.at[c]], rows[b], gsem[b]
            ).wait()

        def add_pe(c, b):
            pe_off = lax.rem(c, PE_PERIOD) * GCHUNK

            def add_row(r, c2):
                for j in range(CHUNKS_PER_ROW):
                    sl = pl.ds(j * LANES, LANES)
                    plsc.addupdate(rows[b].at[r, sl], pe_v[pe_off + r, sl])
                return c2

            lax.fori_loop(0, GCHUNK, add_row, 0, unroll=4)

        def out_start(c, b):
            pltpu.make_async_copy(
                rows[b], out_hbm.at[pl.ds((chunk_base + c) * GCHUNK, GCHUNK)], osem[b]
            ).start()

        def out_wait(b):
            pltpu.make_async_copy(
                rows[b], out_hbm.at[pl.ds(0, GCHUNK)], osem[b]
            ).wait()

        for b in range(LOOKAHEAD):
            gather_start(b, b)

        for c in range(LOOKAHEAD):
            gather_start(c + LOOKAHEAD, (c + LOOKAHEAD) % NBUF)
            gather_wait(c, c % NBUF)
            add_pe(c, c % NBUF)
            out_start(c, c % NBUF)

        def group(g0, carry):
            for b in range(NBUF):
                c = LOOKAHEAD + g0 * NBUF + b
                bslot = (LOOKAHEAD + b + LOOKAHEAD) % NBUF
                pb = (LOOKAHEAD + b) % NBUF
                out_wait(bslot)
                gather_start(c + LOOKAHEAD, bslot)
                gather_wait(c, pb)
                add_pe(c, pb)
                out_start(c, pb)
            return carry

        n_main = (chunks_per_w - 2 * LOOKAHEAD) // NBUF
        lax.fori_loop(0, n_main, group, 0)

        for c in range(chunks_per_w - LOOKAHEAD, chunks_per_w):
            gather_wait(c, c % NBUF)
            add_pe(c, c % NBUF)
            out_start(c, c % NBUF)

        for b in range(NBUF):
            out_wait(b)

    return k


def kernel(sequence, token_table):
    n_chunks = (B * L) // GCHUNK
    seq = sequence.astype(jnp.int32).reshape(n_chunks, GCHUNK)
    pe = _positional_embedding(L, EMBED)
    info = plsc.get_sparse_core_info()
    n_workers = info.num_cores * info.num_subcores
    out = _make_sc_kernel(n_workers)(seq, token_table, pe)
    return out.reshape(B, L, EMBED)
